# baseline (speedup 1.0000x reference)
import functools

import jax
import jax.numpy as jnp
from jax import lax
from jax.experimental import pallas as pl
from jax.experimental.pallas import tpu as pltpu
from jax.experimental.pallas import tpu_sc as plsc

N_CORES = 2
N_SUBCORES = 16
N_WORKERS = N_CORES * N_SUBCORES
CHUNK = 64
BLK = 5


@functools.lru_cache(maxsize=None)
def _make_sc_scatter(n_nodes: int, d_feat: int, n_edges: int):
    eblk = CHUNK * BLK
    assert n_edges % eblk == 0
    assert n_edges // eblk >= N_WORKERS
    assert n_nodes % N_SUBCORES == 0
    assert d_feat % 16 == 0
    n_blocks = n_edges // eblk
    nsl = d_feat // 16
    slab = (n_nodes // N_SUBCORES) // 8 * 8
    tail = n_nodes - slab * N_SUBCORES
    assert slab % 8 == 0 and tail >= 0
    zc = 8
    for cand in (128, 120, 104, 96, 80, 64, 56, 48, 40, 32, 24, 16, 8):
        if slab % cand == 0 and cand <= CHUNK:
            zc = cand
            break
    n_zcopies = slab // zc

    mesh = plsc.VectorSubcoreMesh(core_axis_name="c", subcore_axis_name="s")

    scratch = [
        pltpu.VMEM_SHARED((n_nodes, d_feat), jnp.float32),
        pltpu.VMEM((2 * eblk,), jnp.int32),
        pltpu.VMEM((2 * eblk,), jnp.int32),
        pltpu.VMEM((2 * eblk,), jnp.float32),
    ]
    scratch += [pltpu.VMEM((CHUNK,), jnp.int32) for _ in range(BLK)]
    scratch += [pltpu.VMEM((CHUNK, d_feat), jnp.float32) for _ in range(BLK)]
    scratch += [pltpu.SemaphoreType.DMA for _ in range(2 * BLK)]
    scratch += [pltpu.SemaphoreType.DMA]

    @functools.partial(
        pl.kernel,
        out_type=jax.ShapeDtypeStruct((N_CORES, n_nodes, d_feat), jnp.float32),
        mesh=mesh,
        scratch_types=scratch,
    )
    def sc_scatter(x_hbm, ei_hbm, w_hbm, out_hbm,
                   acc, src_st, dst_st, w_st, *bufs_and_sems):
        dcur = bufs_and_sems[:BLK]
        rows = bufs_and_sems[BLK:2 * BLK]
        gsem = bufs_and_sems[2 * BLK:3 * BLK]
        ssem = bufs_and_sems[3 * BLK:4 * BLK]
        stsem = bufs_and_sems[4 * BLK]
        cid = lax.axis_index("c")
        sid = lax.axis_index("s")
        wid = cid * N_SUBCORES + sid

        z_v = rows[0]
        def zero_body(r, _):
            for j in range(nsl):
                z_v[r, pl.ds(16 * j, 16)] = jnp.zeros((16,), jnp.float32)
            return 0
        lax.fori_loop(0, CHUNK, zero_body, 0)
        base_row = sid * slab
        zcopies = [
            pltpu.make_async_copy(z_v.at[pl.ds(0, zc)],
                                  acc.at[pl.ds(base_row + t * zc, zc)], stsem)
            for t in range(n_zcopies)]
        for c in zcopies:
            c.start()
        for c in zcopies:
            c.wait()
        if tail:
            @pl.when(sid == 0)
            def _zero_tail():
                pltpu.sync_copy(z_v.at[pl.ds(0, tail)],
                                acc.at[pl.ds(slab * N_SUBCORES, tail)])
        plsc.subcore_barrier()

        nblk = (n_blocks - wid + N_WORKERS - 1) // N_WORKERS

        def _stage_copies(g):
            po = lax.rem(g, 2) * eblk
            base = (wid + g * N_WORKERS) * eblk
            return (
                pltpu.make_async_copy(ei_hbm.at[pl.ds(base, eblk)],
                                      src_st.at[pl.ds(po, eblk)], stsem),
                pltpu.make_async_copy(ei_hbm.at[pl.ds(n_edges + base, eblk)],
                                      dst_st.at[pl.ds(po, eblk)], stsem),
                pltpu.make_async_copy(w_hbm.at[pl.ds(base, eblk)],
                                      w_st.at[pl.ds(po, eblk)], stsem),
            )

        def stage_start(g):
            for c in _stage_copies(g):
                c.start()

        def stage_wait(g):
            for c in _stage_copies(g):
                c.wait()

        def start_gather(g, b, buf, sem):
            po = lax.rem(g, 2) * eblk
            idx = src_st.at[pl.ds(po + b * CHUNK, CHUNK)]
            pltpu.async_copy(x_hbm.at[idx], buf, sem)

        stage_start(0)
        stage_wait(0)
        for b in range(BLK):
            start_gather(0, b, rows[b], gsem[b])

        def outer_body(g, _):
            po = lax.rem(g, 2) * eblk
            for b in range(BLK):
                buf = rows[b]

                if b == 0:
                    @pl.when(g + 1 < nblk)
                    def _prefetch():
                        stage_start(g + 1)

                sb = (b + 3) % BLK
                if b < BLK - 3:
                    @pl.when(g >= 1)
                    def _refill_same_block():
                        pltpu.make_async_copy(
                            rows[sb], acc.at[dcur[sb]], ssem[sb]).wait()
                        start_gather(g, sb, rows[sb], gsem[sb])
                else:
                    @pl.when(g + 1 < nblk)
                    def _refill_next_block():
                        if b == BLK - 3:
                            stage_wait(g + 1)
                        pltpu.make_async_copy(
                            rows[sb], acc.at[dcur[sb]], ssem[sb]).wait()
                        start_gather(g + 1, sb, rows[sb], gsem[sb])

                idx_b = src_st.at[pl.ds(po + b * CHUNK, CHUNK)]
                pltpu.make_async_copy(x_hbm.at[idx_b], buf, gsem[b]).wait()

                def mul_body(grp, _):
                    wv16 = w_st[pl.ds(po + b * CHUNK + 16 * grp, 16)]
                    for l in range(16):
                        wvec = jnp.full((16,), wv16[l], jnp.float32)
                        e = 16 * grp + l
                        for j in range(nsl):
                            sl = pl.ds(16 * j, 16)
                            buf[e, sl] = buf[e, sl] * wvec
                    return 0
                lax.fori_loop(0, CHUNK // 16, mul_body, 0)

                for j in range(CHUNK // 16):
                    dcur[b][pl.ds(16 * j, 16)] = (
                        dst_st[pl.ds(po + b * CHUNK + 16 * j, 16)])

                pltpu.async_copy(buf, acc.at[dcur[b]], ssem[b], add=True)
            return 0
        lax.fori_loop(0, nblk, outer_body, 0)

        for b in range(BLK):
            pltpu.make_async_copy(rows[b], acc.at[dcur[b]], ssem[b]).wait()
        plsc.subcore_barrier()

        pltpu.sync_copy(acc.at[pl.ds(base_row, slab)],
                        out_hbm.at[cid, pl.ds(base_row, slab)])
        if tail:
            @pl.when(sid == 0)
            def _write_tail():
                pltpu.sync_copy(acc.at[pl.ds(slab * N_SUBCORES, tail)],
                                out_hbm.at[cid, pl.ds(slab * N_SUBCORES, tail)])

    return sc_scatter


def _mlp_body(p_ref, w1_ref, b1_ref, w2_ref, b2_ref, o_ref):
    agg = p_ref[0] + p_ref[1]
    h = jnp.dot(agg, w1_ref[...], preferred_element_type=jnp.float32)
    h = jnp.maximum(h + b1_ref[...], 0.0)
    o = jnp.dot(h, w2_ref[...], preferred_element_type=jnp.float32)
    o_ref[...] = o + b2_ref[...]


@functools.lru_cache(maxsize=None)
def _make_tc_mlp(n_nodes: int, d_feat: int, d_hid: int):
    rb = 1000 if n_nodes % 1000 == 0 else n_nodes
    grid = (n_nodes // rb,)
    return pl.pallas_call(
        _mlp_body,
        grid=grid,
        in_specs=[
            pl.BlockSpec((N_CORES, rb, d_feat), lambda i: (0, i, 0)),
            pl.BlockSpec((d_feat, d_hid), lambda i: (0, 0)),
            pl.BlockSpec((1, d_hid), lambda i: (0, 0)),
            pl.BlockSpec((d_hid, d_hid), lambda i: (0, 0)),
            pl.BlockSpec((1, d_hid), lambda i: (0, 0)),
        ],
        out_specs=pl.BlockSpec((rb, d_hid), lambda i: (i, 0)),
        out_shape=jax.ShapeDtypeStruct((n_nodes, d_hid), jnp.float32),
    )


def kernel(x, edge_index, edge_weight, W_gnn, b_gnn, W_fc, b_fc):
    n_nodes, d_feat = x.shape
    n_edges = edge_index.shape[1]
    d_hid = W_gnn.shape[1]
    ei_flat = edge_index.astype(jnp.int32).reshape(-1)
    w = edge_weight.astype(jnp.float32)

    partials = _make_sc_scatter(n_nodes, d_feat, n_edges)(x, ei_flat, w)
    out = _make_tc_mlp(n_nodes, d_feat, d_hid)(
        partials, W_gnn, b_gnn.reshape(1, -1), W_fc, b_fc.reshape(1, -1))
    return out


# speedup vs baseline: 1.0051x; 1.0051x over previous
import functools

import jax
import jax.numpy as jnp
from jax import lax
from jax.experimental import pallas as pl
from jax.experimental.pallas import tpu as pltpu
from jax.experimental.pallas import tpu_sc as plsc

N_CORES = 2
N_SUBCORES = 16
N_WORKERS = N_CORES * N_SUBCORES
CHUNK = 64
BLK = 5


@functools.lru_cache(maxsize=None)
def _make_sc_scatter(n_nodes: int, d_feat: int, n_edges: int):
    eblk = CHUNK * BLK
    assert n_edges % eblk == 0
    assert n_edges // eblk >= N_WORKERS
    assert n_nodes % N_SUBCORES == 0
    assert d_feat % 16 == 0
    n_blocks = n_edges // eblk
    nsl = d_feat // 16
    slab = (n_nodes // N_SUBCORES) // 8 * 8
    tail = n_nodes - slab * N_SUBCORES
    assert slab % 8 == 0 and tail >= 0
    zc = 8
    for cand in (128, 120, 104, 96, 80, 64, 56, 48, 40, 32, 24, 16, 8):
        if slab % cand == 0 and cand <= CHUNK:
            zc = cand
            break
    n_zcopies = slab // zc

    mesh = plsc.VectorSubcoreMesh(core_axis_name="c", subcore_axis_name="s")

    scratch = [
        pltpu.VMEM_SHARED((n_nodes, d_feat), jnp.float32),
        pltpu.VMEM((2 * eblk,), jnp.int32),
        pltpu.VMEM((2 * eblk,), jnp.int32),
        pltpu.VMEM((2 * eblk,), jnp.float32),
    ]
    scratch += [pltpu.VMEM((CHUNK,), jnp.int32) for _ in range(BLK)]
    scratch += [pltpu.VMEM((CHUNK, d_feat), jnp.float32) for _ in range(BLK)]
    scratch += [pltpu.SemaphoreType.DMA for _ in range(2 * BLK)]
    scratch += [pltpu.SemaphoreType.DMA]

    @functools.partial(
        pl.kernel,
        out_type=jax.ShapeDtypeStruct((N_CORES, n_nodes, d_feat), jnp.float32),
        mesh=mesh,
        scratch_types=scratch,
    )
    def sc_scatter(x_hbm, ei_hbm, w_hbm, out_hbm,
                   acc, src_st, dst_st, w_st, *bufs_and_sems):
        dcur = bufs_and_sems[:BLK]
        rows = bufs_and_sems[BLK:2 * BLK]
        gsem = bufs_and_sems[2 * BLK:3 * BLK]
        ssem = bufs_and_sems[3 * BLK:4 * BLK]
        stsem = bufs_and_sems[4 * BLK]
        cid = lax.axis_index("c")
        sid = lax.axis_index("s")
        wid = cid * N_SUBCORES + sid

        z_v = rows[0]
        def zero_body(r, _):
            for j in range(nsl):
                z_v[r, pl.ds(16 * j, 16)] = jnp.zeros((16,), jnp.float32)
            return 0
        lax.fori_loop(0, CHUNK, zero_body, 0)
        base_row = sid * slab
        zcopies = [
            pltpu.make_async_copy(z_v.at[pl.ds(0, zc)],
                                  acc.at[pl.ds(base_row + t * zc, zc)], stsem)
            for t in range(n_zcopies)]
        for c in zcopies:
            c.start()
        for c in zcopies:
            c.wait()
        if tail:
            @pl.when(sid == 0)
            def _zero_tail():
                pltpu.sync_copy(z_v.at[pl.ds(0, tail)],
                                acc.at[pl.ds(slab * N_SUBCORES, tail)])
        plsc.subcore_barrier()

        nblk = (n_blocks - wid + N_WORKERS - 1) // N_WORKERS

        def _stage_copies(g):
            po = lax.rem(g, 2) * eblk
            base = (wid + g * N_WORKERS) * eblk
            return (
                pltpu.make_async_copy(ei_hbm.at[pl.ds(base, eblk)],
                                      src_st.at[pl.ds(po, eblk)], stsem),
                pltpu.make_async_copy(ei_hbm.at[pl.ds(n_edges + base, eblk)],
                                      dst_st.at[pl.ds(po, eblk)], stsem),
                pltpu.make_async_copy(w_hbm.at[pl.ds(base, eblk)],
                                      w_st.at[pl.ds(po, eblk)], stsem),
            )

        def stage_start(g):
            for c in _stage_copies(g):
                c.start()

        def stage_wait(g):
            for c in _stage_copies(g):
                c.wait()

        def start_gather(g, b, buf, sem):
            po = lax.rem(g, 2) * eblk
            idx = src_st.at[pl.ds(po + b * CHUNK, CHUNK)]
            pltpu.async_copy(x_hbm.at[idx], buf, sem)

        stage_start(0)
        stage_wait(0)
        for b in range(BLK):
            start_gather(0, b, rows[b], gsem[b])

        def outer_body(g, _):
            po = lax.rem(g, 2) * eblk
            for b in range(BLK):
                buf = rows[b]

                if b == 0:
                    @pl.when(g + 1 < nblk)
                    def _prefetch():
                        stage_start(g + 1)

                sb = (b + 2) % BLK
                if b < BLK - 2:
                    @pl.when(g >= 1)
                    def _refill_same_block():
                        pltpu.make_async_copy(
                            rows[sb], acc.at[dcur[sb]], ssem[sb]).wait()
                        start_gather(g, sb, rows[sb], gsem[sb])
                else:
                    @pl.when(g + 1 < nblk)
                    def _refill_next_block():
                        if b == BLK - 2:
                            stage_wait(g + 1)
                        pltpu.make_async_copy(
                            rows[sb], acc.at[dcur[sb]], ssem[sb]).wait()
                        start_gather(g + 1, sb, rows[sb], gsem[sb])

                idx_b = src_st.at[pl.ds(po + b * CHUNK, CHUNK)]
                pltpu.make_async_copy(x_hbm.at[idx_b], buf, gsem[b]).wait()

                def mul_body(grp, _):
                    wv16 = w_st[pl.ds(po + b * CHUNK + 16 * grp, 16)]
                    for l in range(16):
                        wvec = jnp.full((16,), wv16[l], jnp.float32)
                        e = 16 * grp + l
                        for j in range(nsl):
                            sl = pl.ds(16 * j, 16)
                            buf[e, sl] = buf[e, sl] * wvec
                    return 0
                lax.fori_loop(0, CHUNK // 16, mul_body, 0)

                for j in range(CHUNK // 16):
                    dcur[b][pl.ds(16 * j, 16)] = (
                        dst_st[pl.ds(po + b * CHUNK + 16 * j, 16)])

                pltpu.async_copy(buf, acc.at[dcur[b]], ssem[b], add=True)
            return 0
        lax.fori_loop(0, nblk, outer_body, 0)

        for b in range(BLK):
            pltpu.make_async_copy(rows[b], acc.at[dcur[b]], ssem[b]).wait()
        plsc.subcore_barrier()

        pltpu.sync_copy(acc.at[pl.ds(base_row, slab)],
                        out_hbm.at[cid, pl.ds(base_row, slab)])
        if tail:
            @pl.when(sid == 0)
            def _write_tail():
                pltpu.sync_copy(acc.at[pl.ds(slab * N_SUBCORES, tail)],
                                out_hbm.at[cid, pl.ds(slab * N_SUBCORES, tail)])

    return sc_scatter


def _mlp_body(p_ref, w1_ref, b1_ref, w2_ref, b2_ref, o_ref):
    agg = p_ref[0] + p_ref[1]
    h = jnp.dot(agg, w1_ref[...], preferred_element_type=jnp.float32)
    h = jnp.maximum(h + b1_ref[...], 0.0)
    o = jnp.dot(h, w2_ref[...], preferred_element_type=jnp.float32)
    o_ref[...] = o + b2_ref[...]


@functools.lru_cache(maxsize=None)
def _make_tc_mlp(n_nodes: int, d_feat: int, d_hid: int):
    rb = 1000 if n_nodes % 1000 == 0 else n_nodes
    grid = (n_nodes // rb,)
    return pl.pallas_call(
        _mlp_body,
        grid=grid,
        in_specs=[
            pl.BlockSpec((N_CORES, rb, d_feat), lambda i: (0, i, 0)),
            pl.BlockSpec((d_feat, d_hid), lambda i: (0, 0)),
            pl.BlockSpec((1, d_hid), lambda i: (0, 0)),
            pl.BlockSpec((d_hid, d_hid), lambda i: (0, 0)),
            pl.BlockSpec((1, d_hid), lambda i: (0, 0)),
        ],
        out_specs=pl.BlockSpec((rb, d_hid), lambda i: (i, 0)),
        out_shape=jax.ShapeDtypeStruct((n_nodes, d_hid), jnp.float32),
    )


def kernel(x, edge_index, edge_weight, W_gnn, b_gnn, W_fc, b_fc):
    n_nodes, d_feat = x.shape
    n_edges = edge_index.shape[1]
    d_hid = W_gnn.shape[1]
    ei_flat = edge_index.astype(jnp.int32).reshape(-1)
    w = edge_weight.astype(jnp.float32)

    partials = _make_sc_scatter(n_nodes, d_feat, n_edges)(x, ei_flat, w)
    out = _make_tc_mlp(n_nodes, d_feat, d_hid)(
        partials, W_gnn, b_gnn.reshape(1, -1), W_fc, b_fc.reshape(1, -1))
    return out
